# baseline (speedup 1.0000x reference)
import functools

import jax
import jax.numpy as jnp
from jax import lax
from jax.experimental import pallas as pl
from jax.experimental.pallas import tpu as pltpu
from jax.experimental.pallas import tpu_sc as plsc

NUM_FIXED_ROWS = 50000
EMB_D = 64
NUM_CORES = 2
NUM_SUBCORES = 16
NUM_WORKERS = NUM_CORES * NUM_SUBCORES
CHUNK = 128
LANES = 16
NBUF = 8


def _compact(idx_v, cf_idx, cf_dst, ct_idx, ct_dst, per_w, base):
    iota = lax.iota(jnp.int32, LANES)

    def body(i, carry):
        nf, nt = carry
        v = idx_v[pl.ds(i * LANES, LANES)]
        is_fixed = v < NUM_FIXED_ROWS
        is_tuned = v >= NUM_FIXED_ROWS
        ones = jnp.where(is_fixed, 1, 0)
        incl = plsc.cumsum(ones)
        pos = iota + (base + i * LANES)
        offs_f = nf + incl - 1
        offs_t = nt + iota + 1 - incl - 1
        plsc.store_scatter(cf_idx, [offs_f], v, mask=is_fixed)
        plsc.store_scatter(cf_dst, [offs_f], pos, mask=is_fixed)
        plsc.store_scatter(ct_idx, [offs_t], v - NUM_FIXED_ROWS, mask=is_tuned)
        plsc.store_scatter(ct_dst, [offs_t], pos, mask=is_tuned)
        cnt = jnp.sum(ones)
        return nf + cnt, nt + (LANES - cnt)

    return lax.fori_loop(0, per_w // LANES, body, (jnp.int32(0), jnp.int32(0)))


def _pad_tail(c_idx, c_dst, n):
    iota = lax.iota(jnp.int32, LANES)
    lastv = jnp.zeros((LANES,), jnp.int32) + jnp.maximum(n - 1, 0)
    v_idx = plsc.load_gather(c_idx, [lastv])
    v_dst = plsc.load_gather(c_dst, [lastv])
    start = (n // LANES) * LANES
    for k in range(CHUNK // LANES + 1):
        offs = iota + (start + k * LANES)
        m = offs >= n
        plsc.store_scatter(c_idx, [offs], v_idx, mask=m)
        plsc.store_scatter(c_dst, [offs], v_dst, mask=m)


def _emb_body(idx_hbm, fixed_hbm, tuned_hbm, out_hbm,
              idx_v, cf_idx, cf_dst, ct_idx, ct_dst, rows,
              gi0, gi1, gi2, gi3, gi4, gi5, gi6, gi7,
              sd0, sd1, sd2, sd3, sd4, sd5, sd6, sd7,
              g0, g1, g2, g3, g4, g5, g6, g7,
              s0, s1, s2, s3, s4, s5, s6, s7):
    gidx = (gi0, gi1, gi2, gi3, gi4, gi5, gi6, gi7)
    sdst = (sd0, sd1, sd2, sd3, sd4, sd5, sd6, sd7)
    gsems = (g0, g1, g2, g3, g4, g5, g6, g7)
    ssems = (s0, s1, s2, s3, s4, s5, s6, s7)
    n_total = idx_hbm.shape[0]
    per_w = n_total // NUM_WORKERS
    wid = lax.axis_index("s") * NUM_CORES + lax.axis_index("c")
    base = wid * per_w

    pltpu.sync_copy(idx_hbm.at[pl.ds(base, per_w)], idx_v)

    nf, nt = _compact(idx_v, cf_idx, cf_dst, ct_idx, ct_dst, per_w, base)
    _pad_tail(cf_idx, cf_dst, nf)
    _pad_tail(ct_idx, ct_dst, nt)


    max_groups = (per_w // CHUNK + NBUF - 1) // NBUF

    def run_table(table_hbm, c_idx, c_dst, n):
        n_chunks = (n + CHUNK - 1) // CHUNK

        def stage_idx(i, b):
            for k in range(CHUNK // LANES):
                gidx[b][pl.ds(k * LANES, LANES)] = \
                    c_idx[pl.ds(i * CHUNK + k * LANES, LANES)]
                sdst[b][pl.ds(k * LANES, LANES)] = \
                    c_dst[pl.ds(i * CHUNK + k * LANES, LANES)]

        def group(g, carry):
            for b in range(NBUF):
                i = g * NBUF + b

                @pl.when(jnp.logical_and(i < n_chunks, g > 0))
                def _():
                    pltpu.make_async_copy(
                        rows.at[b], out_hbm.at[pl.ds(0, CHUNK)],
                        ssems[b]).wait()

                @pl.when(i < n_chunks)
                def _():
                    stage_idx(i, b)
                    pltpu.make_async_copy(
                        table_hbm.at[gidx[b]],
                        rows.at[b], gsems[b]).start()
            for b in range(NBUF):
                i = g * NBUF + b

                @pl.when(i < n_chunks)
                def _():
                    pltpu.make_async_copy(
                        table_hbm.at[gidx[b]],
                        rows.at[b], gsems[b]).wait()
                    pltpu.make_async_copy(
                        rows.at[b],
                        out_hbm.at[sdst[b]],
                        ssems[b]).start()
            return carry

        lax.fori_loop(0, max_groups, group, 0)

        rem = ((n_chunks - 1) % NBUF) + 1

        def drain(b):
            @pl.when(jnp.logical_and(n_chunks > 0, b < rem))
            def _():
                pltpu.make_async_copy(rows.at[b], out_hbm.at[pl.ds(0, CHUNK)],
                                      ssems[b]).wait()

        for b in range(NBUF):
            drain(b)

    run_table(fixed_hbm, cf_idx, cf_dst, nf)
    run_table(tuned_hbm, ct_idx, ct_dst, nt)


def _make_run(n_total):
    per_w = n_total // NUM_WORKERS

    mesh = plsc.VectorSubcoreMesh(core_axis_name="c", subcore_axis_name="s")
    return pl.kernel(
        _emb_body,
        out_type=jax.ShapeDtypeStruct((n_total, EMB_D), jnp.float32),
        mesh=mesh,
        scratch_types=[
            pltpu.VMEM((per_w,), jnp.int32),
            pltpu.VMEM((per_w + CHUNK + LANES,), jnp.int32),
            pltpu.VMEM((per_w + CHUNK + LANES,), jnp.int32),
            pltpu.VMEM((per_w + CHUNK + LANES,), jnp.int32),
            pltpu.VMEM((per_w + CHUNK + LANES,), jnp.int32),
            pltpu.VMEM((NBUF, CHUNK, EMB_D), jnp.float32),
        ] + [pltpu.VMEM((CHUNK,), jnp.int32)] * (2 * NBUF) \
          + [pltpu.SemaphoreType.DMA] * (2 * NBUF) + [
        ],
        compiler_params=pltpu.CompilerParams(use_tc_tiling_on_sc=False,
                                             needs_layout_passes=False),
    )

def kernel(input, fixed_tokens, tuned_tokens):
    b, h = input.shape
    hh = h // 2
    run = _make_run(b * hh)
    parts = []
    for piece in (input[:, :hh], input[:, hh:]):
        idx_flat = piece.reshape(b * hh).astype(jnp.int32)
        out = run(idx_flat, fixed_tokens, tuned_tokens)
        parts.append(out.reshape(b, hh, EMB_D))
    return jnp.concatenate(parts, axis=1)


# speedup vs baseline: 1.1114x; 1.1114x over previous
import functools

import jax
import jax.numpy as jnp
from jax import lax
from jax.experimental import pallas as pl
from jax.experimental.pallas import tpu as pltpu
from jax.experimental.pallas import tpu_sc as plsc

NUM_FIXED_ROWS = 50000
EMB_D = 64
NUM_CORES = 2
NUM_SUBCORES = 16
NUM_WORKERS = NUM_CORES * NUM_SUBCORES
CHUNK = 128
LANES = 16
NBUF = 8


def _compact(idx_v, cf_idx, cf_dst, ct_idx, ct_dst, per_w, base):
    iota = lax.iota(jnp.int32, LANES)

    def body(i, carry):
        nf, nt = carry
        v = idx_v[pl.ds(i * LANES, LANES)]
        is_fixed = v < NUM_FIXED_ROWS
        is_tuned = v >= NUM_FIXED_ROWS
        ones = jnp.where(is_fixed, 1, 0)
        incl = plsc.cumsum(ones)
        pos = iota + (base + i * LANES)
        offs_f = nf + incl - 1
        offs_t = nt + iota + 1 - incl - 1
        plsc.store_scatter(cf_idx, [offs_f], v, mask=is_fixed)
        plsc.store_scatter(cf_dst, [offs_f], pos, mask=is_fixed)
        plsc.store_scatter(ct_idx, [offs_t], v - NUM_FIXED_ROWS, mask=is_tuned)
        plsc.store_scatter(ct_dst, [offs_t], pos, mask=is_tuned)
        cnt = jnp.sum(ones)
        return nf + cnt, nt + (LANES - cnt)

    return lax.fori_loop(0, per_w // LANES, body, (jnp.int32(0), jnp.int32(0)))


def _pad_tail(c_idx, c_dst, n):
    iota = lax.iota(jnp.int32, LANES)
    lastv = jnp.zeros((LANES,), jnp.int32) + jnp.maximum(n - 1, 0)
    v_idx = plsc.load_gather(c_idx, [lastv])
    v_dst = plsc.load_gather(c_dst, [lastv])
    start = (n // LANES) * LANES
    for k in range(CHUNK // LANES + 1):
        offs = iota + (start + k * LANES)
        m = offs >= n
        plsc.store_scatter(c_idx, [offs], v_idx, mask=m)
        plsc.store_scatter(c_dst, [offs], v_dst, mask=m)


def _emb_body(idx_hbm, fixed_hbm, tuned_hbm, out_hbm,
              idx_v, cf_idx, cf_dst, ct_idx, ct_dst, rows,
              gi0, gi1, gi2, gi3, gi4, gi5, gi6, gi7,
              sd0, sd1, sd2, sd3, sd4, sd5, sd6, sd7,
              g0, g1, g2, g3, g4, g5, g6, g7,
              s0, s1, s2, s3, s4, s5, s6, s7):
    gidx = (gi0, gi1, gi2, gi3, gi4, gi5, gi6, gi7)
    sdst = (sd0, sd1, sd2, sd3, sd4, sd5, sd6, sd7)
    gsems = (g0, g1, g2, g3, g4, g5, g6, g7)
    ssems = (s0, s1, s2, s3, s4, s5, s6, s7)
    n_total = idx_hbm.shape[0]
    per_w = n_total // NUM_WORKERS
    wid = lax.axis_index("s") * NUM_CORES + lax.axis_index("c")
    base = wid * per_w

    pltpu.sync_copy(idx_hbm.at[pl.ds(base, per_w)], idx_v)

    nf, nt = _compact(idx_v, cf_idx, cf_dst, ct_idx, ct_dst, per_w, base)
    _pad_tail(cf_idx, cf_dst, nf)
    _pad_tail(ct_idx, ct_dst, nt)


    max_groups = (per_w // CHUNK + NBUF - 1) // NBUF

    def run_table(table_hbm, c_idx, c_dst, n):
        n_chunks = (n + CHUNK - 1) // CHUNK

        def stage_idx(i, b):
            for k in range(CHUNK // LANES):
                gidx[b][pl.ds(k * LANES, LANES)] = \
                    c_idx[pl.ds(i * CHUNK + k * LANES, LANES)]
                sdst[b][pl.ds(k * LANES, LANES)] = \
                    c_dst[pl.ds(i * CHUNK + k * LANES, LANES)]

        def group(g, carry):
            for b in range(NBUF):
                i = g * NBUF + b

                @pl.when(jnp.logical_and(i < n_chunks, g > 0))
                def _():
                    pltpu.make_async_copy(
                        rows.at[b], out_hbm.at[pl.ds(0, CHUNK)],
                        ssems[b]).wait()

                @pl.when(i < n_chunks)
                def _():
                    stage_idx(i, b)
                    pltpu.make_async_copy(
                        table_hbm.at[gidx[b]],
                        rows.at[b], gsems[b]).start()
            for b in range(NBUF):
                i = g * NBUF + b

                @pl.when(i < n_chunks)
                def _():
                    pltpu.make_async_copy(
                        table_hbm.at[gidx[b]],
                        rows.at[b], gsems[b]).wait()
                    pltpu.make_async_copy(
                        rows.at[b],
                        out_hbm.at[sdst[b]],
                        ssems[b]).start()
            return carry

        lax.fori_loop(0, max_groups, group, 0)


        def drain(b):
            @pl.when(b < n_chunks)
            def _():
                pltpu.make_async_copy(rows.at[b], out_hbm.at[pl.ds(0, CHUNK)],
                                      ssems[b]).wait()

        for b in range(NBUF):
            drain(b)

    run_table(fixed_hbm, cf_idx, cf_dst, nf)
    run_table(tuned_hbm, ct_idx, ct_dst, nt)


def kernel(input, fixed_tokens, tuned_tokens):
    b, h = input.shape
    n_total = b * h
    idx_flat = input.reshape(n_total).astype(jnp.int32)
    per_w = n_total // NUM_WORKERS

    mesh = plsc.VectorSubcoreMesh(core_axis_name="c", subcore_axis_name="s")
    run = pl.kernel(
        _emb_body,
        out_type=jax.ShapeDtypeStruct((n_total, EMB_D), jnp.float32),
        mesh=mesh,
        scratch_types=[
            pltpu.VMEM((per_w,), jnp.int32),
            pltpu.VMEM((per_w + CHUNK + LANES,), jnp.int32),
            pltpu.VMEM((per_w + CHUNK + LANES,), jnp.int32),
            pltpu.VMEM((per_w + CHUNK + LANES,), jnp.int32),
            pltpu.VMEM((per_w + CHUNK + LANES,), jnp.int32),
            pltpu.VMEM((NBUF, CHUNK, EMB_D), jnp.float32),
        ] + [pltpu.VMEM((CHUNK,), jnp.int32)] * (2 * NBUF) \
          + [pltpu.SemaphoreType.DMA] * (2 * NBUF) + [
        ],
        compiler_params=pltpu.CompilerParams(use_tc_tiling_on_sc=False,
                                             needs_layout_passes=False),
    )
    out = run(idx_flat, fixed_tokens, tuned_tokens)
    return out.reshape(b, h, EMB_D)
